# baseline (speedup 1.0000x reference)
import numpy as np
import jax
import jax.numpy as jnp
from jax.experimental import pallas as pl
from jax.experimental.pallas import tpu as pltpu

_U32 = np.uint32


def _np_threefry2x32(k0, k1, x0, x1):
    ks = [_U32(k0), _U32(k1), _U32(_U32(k0) ^ _U32(k1) ^ _U32(0x1BD11BDA))]
    rots = [[13, 15, 26, 6], [17, 29, 16, 24]]
    x0 = (x0 + ks[0]).astype(np.uint32)
    x1 = (x1 + ks[1]).astype(np.uint32)
    for i in range(5):
        for r in rots[i % 2]:
            x0 = (x0 + x1).astype(np.uint32)
            x1 = ((x1 << _U32(r)) | (x1 >> _U32(32 - r))).astype(np.uint32)
            x1 = (x1 ^ x0).astype(np.uint32)
        x0 = (x0 + ks[(i + 1) % 3]).astype(np.uint32)
        x1 = (x1 + ks[(i + 2) % 3] + _U32(i + 1)).astype(np.uint32)
    return x0, x1


def _np_split(keypair, num=2):
    lo = np.arange(num, dtype=np.uint32)
    hi = np.zeros(num, dtype=np.uint32)
    o0, o1 = _np_threefry2x32(keypair[0], keypair[1], hi, lo)
    return [(int(o0[i]), int(o1[i])) for i in range(num)]


def _key_schedule():
    k1, k2 = _np_split((0, 42))
    out = []
    for k in (k1, k2):
        ku, kf = _np_split(k)
        _, kfb = _np_split(kf)
        out.append((ku, kfb))
    return out


_KEYS = _key_schedule()


def _tf_rounds(k0, k1, x0, x1):
    ks0 = jnp.uint32(k0)
    ks1 = jnp.uint32(k1)
    ks2 = jnp.uint32(k0 ^ k1 ^ 0x1BD11BDA)
    ks = (ks0, ks1, ks2)
    rots = ((13, 15, 26, 6), (17, 29, 16, 24))
    x0 = x0 + ks0
    x1 = x1 + ks1
    for i in range(5):
        for r in rots[i % 2]:
            x0 = x0 + x1
            x1 = (x1 << r) | (x1 >> (32 - r))
            x1 = x1 ^ x0
        x0 = x0 + ks[(i + 1) % 3]
        x1 = x1 + ks[(i + 2) % 3] + jnp.uint32(i + 1)
    return x0, x1


def _layer_kernel(out_f, in_f, ku, kfb):
    ku0, ku1 = ku
    kfb0, kfb1 = kfb

    def body(x_ref, pt_ref, im_ref, o_ref):
        b = pl.program_id(0)
        base_row = jnp.uint32(b) * jnp.uint32(out_f)
        ii = jax.lax.broadcasted_iota(jnp.uint32, (in_f, out_f), 0)
        oo = jax.lax.broadcasted_iota(jnp.uint32, (in_f, out_f), 1)
        lo = (base_row + oo) * jnp.uint32(in_f) + ii
        hi = jnp.zeros((in_f, out_f), jnp.uint32)
        b0, b1 = _tf_rounds(ku0, ku1, hi, lo)
        bits = b0 ^ b1
        fb = (bits >> jnp.uint32(9)) | jnp.uint32(0x3F800000)
        u = jax.lax.bitcast_convert_type(fb, jnp.float32) - jnp.float32(1.0)
        mask = u < pt_ref[...]

        co = jax.lax.broadcasted_iota(jnp.uint32, (1, out_f), 1) + base_row
        f0, f1 = _tf_rounds(kfb0, kfb1, jnp.zeros((1, out_f), jnp.uint32), co)
        fid = (f0 ^ f1) & jnp.uint32(in_f - 1)
        any_row = jnp.any(mask, axis=0, keepdims=True)
        onehot = ii == fid
        mask = mask | (jnp.logical_not(any_row) & onehot)

        im = im_ref[...] != 0
        offs = jnp.where(im, jnp.float32(10.0), jnp.float32(-10.0))
        xcol = x_ref[0, :, :]
        ev = jnp.where(mask, xcol, offs)
        mn = jnp.min(ev, axis=0, keepdims=True)
        mx = jnp.max(ev, axis=0, keepdims=True)
        o_ref[0, :, :] = jnp.where(im, mn, mx)

    return body


def _p_kernel(ct_ref, pt_ref):
    c0 = ct_ref[0, :, :]
    c1 = ct_ref[1, :, :]
    pt_ref[...] = c1 / (c0 + c1)


def _run_layer(x, counts, is_min, keys):
    B = x.shape[0]
    out_f, in_f = counts.shape[0], counts.shape[1]
    ct = jnp.transpose(counts, (2, 1, 0))
    pt = pl.pallas_call(
        _p_kernel,
        out_shape=jax.ShapeDtypeStruct((in_f, out_f), jnp.float32),
    )(ct)
    im = is_min.astype(jnp.int32).reshape(1, out_f)
    xr = x.reshape(B, in_f, 1)
    out = pl.pallas_call(
        _layer_kernel(out_f, in_f, *keys),
        grid=(B,),
        in_specs=[
            pl.BlockSpec((1, in_f, 1), lambda b: (b, 0, 0)),
            pl.BlockSpec((in_f, out_f), lambda b: (0, 0)),
            pl.BlockSpec((1, out_f), lambda b: (0, 0)),
        ],
        out_specs=pl.BlockSpec((1, 1, out_f), lambda b: (b, 0, 0)),
        out_shape=jax.ShapeDtypeStruct((B, 1, out_f), jnp.float32),
        compiler_params=pltpu.CompilerParams(
            dimension_semantics=("parallel",)),
    )(xr, pt, im)
    return out.reshape(B, out_f)


def kernel(x, counts1, counts2, is_min1, is_min2):
    h = _run_layer(x, counts1, is_min1, _KEYS[0])
    y = _run_layer(h, counts2, is_min2, _KEYS[1])
    return y


# speedup vs baseline: 1.4899x; 1.4899x over previous
import numpy as np
import jax
import jax.numpy as jnp
from jax.experimental import pallas as pl
from jax.experimental.pallas import tpu as pltpu

_U32 = np.uint32


def _np_threefry2x32(k0, k1, x0, x1):
    ks = [_U32(k0), _U32(k1), _U32(_U32(k0) ^ _U32(k1) ^ _U32(0x1BD11BDA))]
    rots = [[13, 15, 26, 6], [17, 29, 16, 24]]
    x0 = (x0 + ks[0]).astype(np.uint32)
    x1 = (x1 + ks[1]).astype(np.uint32)
    for i in range(5):
        for r in rots[i % 2]:
            x0 = (x0 + x1).astype(np.uint32)
            x1 = ((x1 << _U32(r)) | (x1 >> _U32(32 - r))).astype(np.uint32)
            x1 = (x1 ^ x0).astype(np.uint32)
        x0 = (x0 + ks[(i + 1) % 3]).astype(np.uint32)
        x1 = (x1 + ks[(i + 2) % 3] + _U32(i + 1)).astype(np.uint32)
    return x0, x1


def _np_split(keypair, num=2):
    lo = np.arange(num, dtype=np.uint32)
    hi = np.zeros(num, dtype=np.uint32)
    o0, o1 = _np_threefry2x32(keypair[0], keypair[1], hi, lo)
    return [(int(o0[i]), int(o1[i])) for i in range(num)]


def _key_schedule():
    k1, k2 = _np_split((0, 42))
    out = []
    for k in (k1, k2):
        ku, kf = _np_split(k)
        _, kfb = _np_split(kf)
        out.append((ku, kfb))
    return out


_KEYS = _key_schedule()


def _tf_rounds(k0, k1, x0, x1):
    ks0 = jnp.uint32(k0)
    ks1 = jnp.uint32(k1)
    ks2 = jnp.uint32(k0 ^ k1 ^ 0x1BD11BDA)
    ks = (ks0, ks1, ks2)
    rots = ((13, 15, 26, 6), (17, 29, 16, 24))
    x0 = x0 + ks0
    x1 = x1 + ks1
    for i in range(5):
        for r in rots[i % 2]:
            x0 = x0 + x1
            x1 = (x1 << r) | (x1 >> (32 - r))
            x1 = x1 ^ x0
        x0 = x0 + ks[(i + 1) % 3]
        x1 = x1 + ks[(i + 2) % 3] + jnp.uint32(i + 1)
    return x0, x1


_CHUNK = 16


def _layer_kernel(out_f, in_f, ku, kfb):
    ku0, ku1 = ku
    kfb0, kfb1 = kfb
    C = _CHUNK
    n_chunks = in_f // C

    def body(x_ref, th_ref, im_ref, o_ref):
        b = pl.program_id(0)
        base_row = jnp.uint32(b) * jnp.uint32(out_f)
        ii = jax.lax.broadcasted_iota(jnp.uint32, (C, out_f), 0)
        oo = jax.lax.broadcasted_iota(jnp.uint32, (C, out_f), 1)
        row_term = (base_row + oo[0:1, :]) * jnp.uint32(in_f)

        co = jax.lax.broadcasted_iota(jnp.uint32, (1, out_f), 1) + base_row
        f0, f1 = _tf_rounds(kfb0, kfb1, jnp.zeros((1, out_f), jnp.uint32), co)
        fid = (f0 ^ f1) & jnp.uint32(in_f - 1)

        im = im_ref[...] != 0
        offs = jnp.where(im, jnp.float32(10.0), jnp.float32(-10.0))

        def step(j, carry):
            mn_a, mx_a, any_a, f_a = carry
            jc = jnp.uint32(j) * jnp.uint32(C)
            lo = row_term + (ii + jc)
            hi = jnp.zeros((C, out_f), jnp.uint32)
            b0, b1 = _tf_rounds(ku0, ku1, hi, lo)
            m = ((b0 ^ b1) >> jnp.uint32(9)) < th_ref[pl.dslice(j * C, C), :]
            xc = x_ref[0, pl.dslice(j * C, C), :]
            ev = jnp.where(m, xc, offs)
            mn_a = jnp.minimum(mn_a, jnp.min(ev, axis=0, keepdims=True))
            mx_a = jnp.maximum(mx_a, jnp.max(ev, axis=0, keepdims=True))
            any_a = jnp.where(jnp.any(m, axis=0, keepdims=True),
                              jnp.int32(1), any_a)
            oh = (ii + jc) == fid
            f_a = f_a + jnp.sum(jnp.where(oh, xc, jnp.float32(0.0)),
                                axis=0, keepdims=True)
            return mn_a, mx_a, any_a, f_a

        init = (jnp.full((1, out_f), 10.0, jnp.float32),
                jnp.full((1, out_f), -10.0, jnp.float32),
                jnp.zeros((1, out_f), jnp.int32),
                jnp.zeros((1, out_f), jnp.float32))
        mn_a, mx_a, any_a, f_a = jax.lax.fori_loop(0, n_chunks, step, init)

        res = jnp.where(im, mn_a, mx_a)
        fres = jnp.where(im, jnp.minimum(f_a, jnp.float32(10.0)),
                         jnp.maximum(f_a, jnp.float32(-10.0)))
        o_ref[0, :, :] = jnp.where(any_a != 0, res, fres)

    return body


def _p_kernel(ct_ref, th_ref):
    c0 = ct_ref[0, :, :]
    c1 = ct_ref[1, :, :]
    p = c1 / (c0 + c1)
    th_ref[...] = jnp.ceil(p * jnp.float32(8388608.0)).astype(jnp.uint32)


def _run_layer(x, counts, is_min, keys):
    B = x.shape[0]
    out_f, in_f = counts.shape[0], counts.shape[1]
    ct = jnp.transpose(counts, (2, 1, 0))
    pt = pl.pallas_call(
        _p_kernel,
        out_shape=jax.ShapeDtypeStruct((in_f, out_f), jnp.uint32),
    )(ct)
    im = is_min.astype(jnp.int32).reshape(1, out_f)
    xr = x.reshape(B, in_f, 1)
    out = pl.pallas_call(
        _layer_kernel(out_f, in_f, *keys),
        grid=(B,),
        in_specs=[
            pl.BlockSpec((1, in_f, 1), lambda b: (b, 0, 0)),
            pl.BlockSpec((in_f, out_f), lambda b: (0, 0)),
            pl.BlockSpec((1, out_f), lambda b: (0, 0)),
        ],
        out_specs=pl.BlockSpec((1, 1, out_f), lambda b: (b, 0, 0)),
        out_shape=jax.ShapeDtypeStruct((B, 1, out_f), jnp.float32),
        compiler_params=pltpu.CompilerParams(
            dimension_semantics=("parallel",)),
    )(xr, pt, im)
    return out.reshape(B, out_f)


def kernel(x, counts1, counts2, is_min1, is_min2):
    h = _run_layer(x, counts1, is_min1, _KEYS[0])
    y = _run_layer(h, counts2, is_min2, _KEYS[1])
    return y
